# baseline (speedup 1.0000x reference)
import functools

import jax
import jax.numpy as jnp
from jax import lax
from jax.experimental import pallas as pl
from jax.experimental.pallas import tpu as pltpu
from jax.experimental.pallas import tpu_sc as plsc

NC = 2
NS = 16
NW = NC * NS
LANES = 16
CHUNK = 128


@functools.lru_cache(maxsize=None)
def _make_sc_kernel(B, V, D, b_per_w):
    assert D == 2 * LANES
    n_idx = b_per_w * D
    n_chunks = n_idx // CHUNK
    mesh = plsc.VectorSubcoreMesh(core_axis_name="c", subcore_axis_name="s")

    @functools.partial(
        pl.kernel,
        out_type=jax.ShapeDtypeStruct((B,), jnp.float32),
        mesh=mesh,
        scratch_types=[
            pltpu.VMEM((b_per_w,), jnp.int32),
            pltpu.VMEM((b_per_w,), jnp.int32),
            pltpu.VMEM((n_chunks, CHUNK), jnp.int32),
            pltpu.VMEM((n_chunks, CHUNK), jnp.int32),
            pltpu.VMEM((n_idx,), jnp.float32),
            pltpu.VMEM((n_idx,), jnp.float32),
            pltpu.VMEM((b_per_w,), jnp.float32),
            pltpu.SemaphoreType.DMA,
            pltpu.SemaphoreType.DMA,
        ],
        compiler_params=pltpu.CompilerParams(needs_layout_passes=False),
    )
    def k(uids_hbm, iids_hbm, ut_hbm, it_hbm, out_hbm,
          uidx_v, iidx_v, uoff_v, ioff_v, uval_v, ival_v, out_v,
          sem_u, sem_i):
        wid = lax.axis_index("s") * NC + lax.axis_index("c")
        base = wid * b_per_w

        pltpu.sync_copy(uids_hbm.at[pl.ds(base, b_per_w)], uidx_v)
        pltpu.sync_copy(iids_hbm.at[pl.ds(base, b_per_w)], iidx_v)

        lane = lax.iota(jnp.int32, LANES)

        def build(g, carry):
            uvec = uidx_v[pl.ds(g * LANES, LANES)]
            ivec = iidx_v[pl.ds(g * LANES, LANES)]
            for t in range(LANES):
                j = g * LANES + t
                for h in range(D // LANES):
                    off = (lane + h * LANES) * V + uvec[t]
                    flat = j * D + h * LANES
                    uoff_v[flat // CHUNK, pl.ds(flat % CHUNK, LANES)] = off
                    off = (lane + h * LANES) * V + ivec[t]
                    ioff_v[flat // CHUNK, pl.ds(flat % CHUNK, LANES)] = off
            return carry
        lax.fori_loop(0, b_per_w // LANES, build, 0)

        for c in range(n_chunks):
            pltpu.async_copy(
                ut_hbm.at[uoff_v.at[c]],
                uval_v.at[pl.ds(c * CHUNK, CHUNK)], sem_u)
            pltpu.async_copy(
                it_hbm.at[ioff_v.at[c]],
                ival_v.at[pl.ds(c * CHUNK, CHUNK)], sem_i)
        pltpu.make_async_copy(
            ut_hbm.at[pl.ds(0, n_idx)], uval_v, sem_u).wait()
        pltpu.make_async_copy(
            it_hbm.at[pl.ds(0, n_idx)], ival_v, sem_i).wait()

        def group(g, carry):
            def row(i, acc):
                j = g * LANES + i
                u0 = uval_v[pl.ds(j * D, LANES)]
                u1 = uval_v[pl.ds(j * D + LANES, LANES)]
                i0 = ival_v[pl.ds(j * D, LANES)]
                i1 = ival_v[pl.ds(j * D + LANES, LANES)]
                s = jnp.sum(u0 * i0 + u1 * i1)
                return jnp.where(lane == i, s, acc)
            acc = lax.fori_loop(0, LANES, row, jnp.zeros((LANES,), jnp.float32))
            out_v[pl.ds(g * LANES, LANES)] = acc
            return carry
        lax.fori_loop(0, b_per_w // LANES, group, 0)

        pltpu.sync_copy(out_v, out_hbm.at[pl.ds(base, b_per_w)])

    return k


def kernel(user_ids, item_ids, user_table, item_table):
    B = user_ids.shape[0]
    V, D = user_table.shape
    b_per_w = B // NW
    k = _make_sc_kernel(B, V, D, b_per_w)
    ut_flat = user_table.T.reshape(V * D)
    it_flat = item_table.T.reshape(V * D)
    return k(user_ids, item_ids, ut_flat, it_flat)
